# Initial kernel scaffold; baseline (speedup 1.0000x reference)
import jax
import jax.numpy as jnp
from jax.experimental import pallas as pl


def kernel(x, edge_index, Wl0, bl0, Wr0, Wl1, bl1, Wr1, W1, b1, W2, b2, W3, b3):
    raise NotImplementedError("write your pallas kernel here")


# speedup vs baseline: 3.0313x; 3.0313x over previous
import functools

import jax
import jax.numpy as jnp
from jax import lax
from jax.experimental import pallas as pl
from jax.experimental.pallas import tpu as pltpu
from jax.experimental.pallas import tpu_sc as plsc

N_NODES = 10000
N_EDGES = 320000
NT = 16
SUB = 128
CH0 = 80
CH1 = 160
BLK = 16
E_PAD = 2 * NT * CH0 * SUB
RD = 640
ACC_ROWS = NT * RD

_MESH = dict(core_axis_name="c", subcore_axis_name="s")


def _acc_out():
    return jax.ShapeDtypeStruct((ACC_ROWS, 128), jnp.float32)


@functools.cache
def _agg_edge_split():
    scratch = [
        pltpu.VMEM((BLK, SUB), jnp.int32),
        pltpu.VMEM((BLK, SUB), jnp.int32),
        pltpu.VMEM((SUB, 128), jnp.float32),
        pltpu.VMEM_SHARED((ACC_ROWS, 128), jnp.float32),
        pltpu.SemaphoreType.DMA,
    ]

    def body(tab, src_hbm, dst_hbm, zrow_hbm, acc_a_hbm, acc_b_hbm,
             src_buf, dst_buf, rows, acc, sem):
        c = lax.axis_index("c")
        s = lax.axis_index("s")
        w = c * NT + s

        pltpu.sync_copy(zrow_hbm, acc.at[pl.ds(s * RD, RD)])
        plsc.subcore_barrier()

        def blk_it(b, carry):
            base = w * CH0 + b * BLK
            pltpu.sync_copy(src_hbm.at[pl.ds(base, BLK)], src_buf)
            pltpu.sync_copy(dst_hbm.at[pl.ds(base, BLK)], dst_buf)

            def it(j, carry2):
                pltpu.async_copy(tab.at[src_buf.at[j]], rows, sem).wait()
                pltpu.sync_copy(rows, acc.at[dst_buf.at[j]], add=True)
                return carry2
            lax.fori_loop(0, BLK, it, 0)
            return carry
        lax.fori_loop(0, CH0 // BLK, blk_it, 0)

        plsc.subcore_barrier()

        @pl.when(c == 0)
        def _():
            pltpu.sync_copy(acc.at[pl.ds(s * RD, RD)],
                            acc_a_hbm.at[pl.ds(s * RD, RD)])

        @pl.when(c == 1)
        def _():
            pltpu.sync_copy(acc.at[pl.ds(s * RD, RD)],
                            acc_b_hbm.at[pl.ds(s * RD, RD)])

    return pl.kernel(body, mesh=plsc.VectorSubcoreMesh(**_MESH),
                     out_type=[_acc_out(), _acc_out()],
                     scratch_types=scratch)


@functools.cache
def _count_edges():
    scratch = [
        pltpu.VMEM((BLK, SUB), jnp.int32),
        pltpu.VMEM((SUB, 128), jnp.float32),
        pltpu.VMEM_SHARED((ACC_ROWS, 128), jnp.float32),
    ]

    def body(dst_hbm, zrow_hbm, ones_hbm, cnt_a_hbm, cnt_b_hbm,
             dst_buf, ones_buf, acc):
        c = lax.axis_index("c")
        s = lax.axis_index("s")
        w = c * NT + s

        pltpu.sync_copy(zrow_hbm, acc.at[pl.ds(s * RD, RD)])
        pltpu.sync_copy(ones_hbm, ones_buf)
        plsc.subcore_barrier()

        def blk_it(b, carry):
            base = w * CH0 + b * BLK
            pltpu.sync_copy(dst_hbm.at[pl.ds(base, BLK)], dst_buf)

            def it(j, carry2):
                pltpu.sync_copy(ones_buf, acc.at[dst_buf.at[j]], add=True)
                return carry2
            lax.fori_loop(0, BLK, it, 0)
            return carry
        lax.fori_loop(0, CH0 // BLK, blk_it, 0)

        plsc.subcore_barrier()

        @pl.when(c == 0)
        def _():
            pltpu.sync_copy(acc.at[pl.ds(s * RD, RD)],
                            cnt_a_hbm.at[pl.ds(s * RD, RD)])

        @pl.when(c == 1)
        def _():
            pltpu.sync_copy(acc.at[pl.ds(s * RD, RD)],
                            cnt_b_hbm.at[pl.ds(s * RD, RD)])

    return pl.kernel(body, mesh=plsc.VectorSubcoreMesh(**_MESH),
                     out_type=[_acc_out(), _acc_out()],
                     scratch_types=scratch)


@functools.cache
def _agg_feat_split():
    scratch = [
        pltpu.VMEM((BLK, SUB), jnp.int32),
        pltpu.VMEM((BLK, SUB), jnp.int32),
        pltpu.VMEM((SUB, 128), jnp.float32),
        pltpu.VMEM_SHARED((ACC_ROWS, 128), jnp.float32),
        pltpu.SemaphoreType.DMA,
    ]

    def body(tab_lo, tab_hi, src_hbm, dst_hbm, zrow_hbm,
             agg_lo_hbm, agg_hi_hbm,
             src_buf, dst_buf, rows, acc, sem):
        c = lax.axis_index("c")
        s = lax.axis_index("s")

        pltpu.sync_copy(zrow_hbm, acc.at[pl.ds(s * RD, RD)])
        plsc.subcore_barrier()

        def edge_loop(tab):
            def blk_it(b, carry):
                base = s * CH1 + b * BLK
                pltpu.sync_copy(src_hbm.at[pl.ds(base, BLK)], src_buf)
                pltpu.sync_copy(dst_hbm.at[pl.ds(base, BLK)], dst_buf)

                def it(j, carry2):
                    pltpu.async_copy(tab.at[src_buf.at[j]], rows,
                                     sem).wait()
                    pltpu.sync_copy(rows, acc.at[dst_buf.at[j]], add=True)
                    return carry2
                lax.fori_loop(0, BLK, it, 0)
                return carry
            lax.fori_loop(0, CH1 // BLK, blk_it, 0)

        @pl.when(c == 0)
        def _():
            edge_loop(tab_lo)

        @pl.when(c == 1)
        def _():
            edge_loop(tab_hi)

        plsc.subcore_barrier()

        @pl.when(c == 0)
        def _():
            pltpu.sync_copy(acc.at[pl.ds(s * RD, RD)],
                            agg_lo_hbm.at[pl.ds(s * RD, RD)])

        @pl.when(c == 1)
        def _():
            pltpu.sync_copy(acc.at[pl.ds(s * RD, RD)],
                            agg_hi_hbm.at[pl.ds(s * RD, RD)])

    return pl.kernel(body, mesh=plsc.VectorSubcoreMesh(**_MESH),
                     out_type=[_acc_out(), _acc_out()],
                     scratch_types=scratch)


def _tc0_body(aa_ref, ab_ref, ca_ref, cb_ref, x_ref, wl_ref, bl_ref, wr_ref,
              olo_ref, ohi_ref):
    cnt = ca_ref[:, 0:1] + cb_ref[:, 0:1]
    r = 1.0 / jnp.maximum(cnt, 1.0)
    mean = (aa_ref[...] + ab_ref[...]) * r
    h = jnp.dot(mean, wl_ref[...], preferred_element_type=jnp.float32)
    h += jnp.dot(x_ref[...], wr_ref[...], preferred_element_type=jnp.float32)
    h = jnp.maximum(h + bl_ref[...], 0.0)
    olo_ref[...] = h[:, :128]
    ohi_ref[...] = h[:, 128:]


def _tc1_body(al_ref, ah_ref, ca_ref, cb_ref, hlo_ref, hhi_ref, wl_ref,
              bl_ref, wr_ref, w1_ref, b1_ref, w2_ref, b2_ref, w3_ref, b3_ref,
              o_ref):
    cnt = ca_ref[:, 0:1] + cb_ref[:, 0:1]
    r = 1.0 / jnp.maximum(cnt, 1.0)
    wl = wl_ref[...]
    wr = wr_ref[...]
    h = jnp.dot(al_ref[...] * r, wl[:128], preferred_element_type=jnp.float32)
    h += jnp.dot(ah_ref[...] * r, wl[128:], preferred_element_type=jnp.float32)
    h += jnp.dot(hlo_ref[...], wr[:128], preferred_element_type=jnp.float32)
    h += jnp.dot(hhi_ref[...], wr[128:], preferred_element_type=jnp.float32)
    h = jnp.maximum(h + bl_ref[...], 0.0)
    h = jnp.maximum(jnp.dot(h, w1_ref[...],
                            preferred_element_type=jnp.float32) + b1_ref[...],
                    0.0)
    h = jnp.maximum(jnp.dot(h, w2_ref[...],
                            preferred_element_type=jnp.float32) + b2_ref[...],
                    0.0)
    o_ref[...] = jnp.dot(h, w3_ref[...],
                         preferred_element_type=jnp.float32) + b3_ref[...]


_R = 1000


def _row_spec(w):
    return pl.BlockSpec((_R, w), lambda i: (i, 0))


def _full_spec(shape):
    return pl.BlockSpec(shape, lambda i: (0,) * len(shape))


def _tc0(aa, ab, ca, cb, x, wl, bl, wr):
    return pl.pallas_call(
        _tc0_body,
        grid=(N_NODES // _R,),
        in_specs=[
            _row_spec(128), _row_spec(128), _row_spec(128), _row_spec(128),
            _row_spec(128),
            _full_spec((128, 256)), _full_spec((1, 256)),
            _full_spec((128, 256)),
        ],
        out_specs=[_row_spec(128), _row_spec(128)],
        out_shape=[jax.ShapeDtypeStruct((N_NODES, 128), jnp.float32)] * 2,
    )(aa, ab, ca, cb, x, wl, bl, wr)


def _tc1(al, ah, ca, cb, hlo, hhi, wl, bl, wr, w1, b1, w2, b2, w3, b3):
    return pl.pallas_call(
        _tc1_body,
        grid=(N_NODES // _R,),
        in_specs=[
            _row_spec(128), _row_spec(128), _row_spec(128), _row_spec(128),
            _row_spec(128), _row_spec(128),
            _full_spec((256, 256)), _full_spec((1, 256)),
            _full_spec((256, 256)),
            _full_spec((256, 256)), _full_spec((1, 256)),
            _full_spec((256, 256)), _full_spec((1, 256)),
            _full_spec((256, 128)), _full_spec((1, 128)),
        ],
        out_specs=_row_spec(128),
        out_shape=jax.ShapeDtypeStruct((N_NODES, 128), jnp.float32),
    )(al, ah, ca, cb, hlo, hhi, wl, bl, wr, w1, b1, w2, b2, w3, b3)


def kernel(x, edge_index, Wl0, bl0, Wr0, Wl1, bl1, Wr1, W1, b1, W2, b2,
           W3, b3):
    i32 = jnp.int32
    src = edge_index[0].astype(i32)
    dst = edge_index[1].astype(i32)
    pad = E_PAD - N_EDGES
    src_p = jnp.concatenate([src, jnp.zeros((pad,), i32)]).reshape(
        NT * CH1, SUB)
    dst_p = jnp.concatenate([dst, jnp.full((pad,), N_NODES, i32)]).reshape(
        NT * CH1, SUB)

    zrow = jnp.zeros((RD, 128), jnp.float32)
    ones = jnp.ones((SUB, 128), jnp.float32)

    acc_a, acc_b = _agg_edge_split()(x, src_p, dst_p, zrow)
    cnt_a, cnt_b = _count_edges()(dst_p, zrow, ones)
    ca = cnt_a[:N_NODES]
    cb = cnt_b[:N_NODES]

    h_lo, h_hi = _tc0(acc_a[:N_NODES], acc_b[:N_NODES], ca, cb, x,
                      Wl0, bl0.reshape(1, -1), Wr0)

    agg1_lo, agg1_hi = _agg_feat_split()(h_lo, h_hi, src_p, dst_p, zrow)

    w3p = jnp.pad(W3, ((0, 0), (0, 128 - W3.shape[1])))
    b3p = jnp.pad(b3, (0, 128 - b3.shape[0])).reshape(1, -1)
    out = _tc1(agg1_lo[:N_NODES], agg1_hi[:N_NODES], ca, cb, h_lo, h_hi,
               Wl1, bl1.reshape(1, -1), Wr1, W1, b1.reshape(1, -1),
               W2, b2.reshape(1, -1), w3p, b3p)
    return out[:, :40]


# speedup vs baseline: 3.5004x; 1.1548x over previous
import functools

import jax
import jax.numpy as jnp
from jax import lax
from jax.experimental import pallas as pl
from jax.experimental.pallas import tpu as pltpu
from jax.experimental.pallas import tpu_sc as plsc

N_NODES = 10000
N_EDGES = 320000
NT = 16
SUB = 128
CH0 = 80
CH1 = 160
BLK = 40
E_PAD = 2 * NT * CH0 * SUB
RD = 640
ACC_ROWS = NT * RD

_MESH = dict(core_axis_name="c", subcore_axis_name="s")


def _pipelined_agg(tab, src_hbm, dst_hbm, chunk0, nblk,
                   src_buf, dst_buf, rows0, rows1, acc, sem0, sem1):
    def blk_it(b, carry):
        base = chunk0 + b * BLK
        pltpu.sync_copy(src_hbm.at[pl.ds(base, BLK)], src_buf)
        pltpu.sync_copy(dst_hbm.at[pl.ds(base, BLK)], dst_buf)
        pltpu.async_copy(tab.at[src_buf.at[0]], rows0, sem0)

        def pair_it(t, carry2):
            j0 = 2 * t
            j1 = j0 + 1
            j2 = j0 + 2
            pltpu.async_copy(tab.at[src_buf.at[j1]], rows1, sem1)
            pltpu.make_async_copy(tab.at[src_buf.at[j0]], rows0,
                                  sem0).wait()
            pltpu.sync_copy(rows0, acc.at[dst_buf.at[j0]], add=True)

            @pl.when(j2 < BLK)
            def _():
                pltpu.async_copy(tab.at[src_buf.at[j2]], rows0, sem0)

            pltpu.make_async_copy(tab.at[src_buf.at[j1]], rows1,
                                  sem1).wait()
            pltpu.sync_copy(rows1, acc.at[dst_buf.at[j1]], add=True)
            return carry2
        lax.fori_loop(0, BLK // 2, pair_it, 0)
        return carry
    lax.fori_loop(0, nblk, blk_it, 0)


def _acc_out():
    return jax.ShapeDtypeStruct((ACC_ROWS, 128), jnp.float32)


@functools.cache
def _agg_edge_split():
    scratch = [
        pltpu.VMEM((BLK, SUB), jnp.int32),
        pltpu.VMEM((BLK, SUB), jnp.int32),
        pltpu.VMEM((SUB, 128), jnp.float32),
        pltpu.VMEM((SUB, 128), jnp.float32),
        pltpu.VMEM_SHARED((ACC_ROWS, 128), jnp.float32),
        pltpu.SemaphoreType.DMA,
        pltpu.SemaphoreType.DMA,
    ]

    def body(tab, src_hbm, dst_hbm, zrow_hbm, acc_a_hbm, acc_b_hbm,
             src_buf, dst_buf, rows0, rows1, acc, sem0, sem1):
        c = lax.axis_index("c")
        s = lax.axis_index("s")
        w = c * NT + s

        pltpu.sync_copy(zrow_hbm, acc.at[pl.ds(s * RD, RD)])
        plsc.subcore_barrier()

        _pipelined_agg(tab, src_hbm, dst_hbm, w * CH0, CH0 // BLK,
                       src_buf, dst_buf, rows0, rows1, acc, sem0, sem1)

        plsc.subcore_barrier()

        @pl.when(c == 0)
        def _():
            pltpu.sync_copy(acc.at[pl.ds(s * RD, RD)],
                            acc_a_hbm.at[pl.ds(s * RD, RD)])

        @pl.when(c == 1)
        def _():
            pltpu.sync_copy(acc.at[pl.ds(s * RD, RD)],
                            acc_b_hbm.at[pl.ds(s * RD, RD)])

    return pl.kernel(body, mesh=plsc.VectorSubcoreMesh(**_MESH),
                     out_type=[_acc_out(), _acc_out()],
                     scratch_types=scratch)


@functools.cache
def _count_edges():
    scratch = [
        pltpu.VMEM((BLK, SUB), jnp.int32),
        pltpu.VMEM((SUB, 128), jnp.float32),
        pltpu.VMEM_SHARED((ACC_ROWS, 128), jnp.float32),
    ]

    def body(dst_hbm, zrow_hbm, ones_hbm, cnt_a_hbm, cnt_b_hbm,
             dst_buf, ones_buf, acc):
        c = lax.axis_index("c")
        s = lax.axis_index("s")
        w = c * NT + s

        pltpu.sync_copy(zrow_hbm, acc.at[pl.ds(s * RD, RD)])
        pltpu.sync_copy(ones_hbm, ones_buf)
        plsc.subcore_barrier()

        def blk_it(b, carry):
            base = w * CH0 + b * BLK
            pltpu.sync_copy(dst_hbm.at[pl.ds(base, BLK)], dst_buf)

            def it(j, carry2):
                pltpu.sync_copy(ones_buf, acc.at[dst_buf.at[j]], add=True)
                return carry2
            lax.fori_loop(0, BLK, it, 0)
            return carry
        lax.fori_loop(0, CH0 // BLK, blk_it, 0)

        plsc.subcore_barrier()

        @pl.when(c == 0)
        def _():
            pltpu.sync_copy(acc.at[pl.ds(s * RD, RD)],
                            cnt_a_hbm.at[pl.ds(s * RD, RD)])

        @pl.when(c == 1)
        def _():
            pltpu.sync_copy(acc.at[pl.ds(s * RD, RD)],
                            cnt_b_hbm.at[pl.ds(s * RD, RD)])

    return pl.kernel(body, mesh=plsc.VectorSubcoreMesh(**_MESH),
                     out_type=[_acc_out(), _acc_out()],
                     scratch_types=scratch)


@functools.cache
def _agg_feat_split():
    scratch = [
        pltpu.VMEM((BLK, SUB), jnp.int32),
        pltpu.VMEM((BLK, SUB), jnp.int32),
        pltpu.VMEM((SUB, 128), jnp.float32),
        pltpu.VMEM((SUB, 128), jnp.float32),
        pltpu.VMEM_SHARED((ACC_ROWS, 128), jnp.float32),
        pltpu.SemaphoreType.DMA,
        pltpu.SemaphoreType.DMA,
    ]

    def body(tab_lo, tab_hi, src_hbm, dst_hbm, zrow_hbm,
             agg_lo_hbm, agg_hi_hbm,
             src_buf, dst_buf, rows0, rows1, acc, sem0, sem1):
        c = lax.axis_index("c")
        s = lax.axis_index("s")

        pltpu.sync_copy(zrow_hbm, acc.at[pl.ds(s * RD, RD)])
        plsc.subcore_barrier()

        @pl.when(c == 0)
        def _():
            _pipelined_agg(tab_lo, src_hbm, dst_hbm, s * CH1, CH1 // BLK,
                           src_buf, dst_buf, rows0, rows1, acc, sem0, sem1)

        @pl.when(c == 1)
        def _():
            _pipelined_agg(tab_hi, src_hbm, dst_hbm, s * CH1, CH1 // BLK,
                           src_buf, dst_buf, rows0, rows1, acc, sem0, sem1)

        plsc.subcore_barrier()

        @pl.when(c == 0)
        def _():
            pltpu.sync_copy(acc.at[pl.ds(s * RD, RD)],
                            agg_lo_hbm.at[pl.ds(s * RD, RD)])

        @pl.when(c == 1)
        def _():
            pltpu.sync_copy(acc.at[pl.ds(s * RD, RD)],
                            agg_hi_hbm.at[pl.ds(s * RD, RD)])

    return pl.kernel(body, mesh=plsc.VectorSubcoreMesh(**_MESH),
                     out_type=[_acc_out(), _acc_out()],
                     scratch_types=scratch)


def _tc0_body(aa_ref, ab_ref, ca_ref, cb_ref, x_ref, wl_ref, bl_ref, wr_ref,
              olo_ref, ohi_ref):
    cnt = ca_ref[:, 0:1] + cb_ref[:, 0:1]
    r = 1.0 / jnp.maximum(cnt, 1.0)
    mean = (aa_ref[...] + ab_ref[...]) * r
    h = jnp.dot(mean, wl_ref[...], preferred_element_type=jnp.float32)
    h += jnp.dot(x_ref[...], wr_ref[...], preferred_element_type=jnp.float32)
    h = jnp.maximum(h + bl_ref[...], 0.0)
    olo_ref[...] = h[:, :128]
    ohi_ref[...] = h[:, 128:]


def _tc1_body(al_ref, ah_ref, ca_ref, cb_ref, hlo_ref, hhi_ref, wl_ref,
              bl_ref, wr_ref, w1_ref, b1_ref, w2_ref, b2_ref, w3_ref, b3_ref,
              o_ref):
    cnt = ca_ref[:, 0:1] + cb_ref[:, 0:1]
    r = 1.0 / jnp.maximum(cnt, 1.0)
    wl = wl_ref[...]
    wr = wr_ref[...]
    h = jnp.dot(al_ref[...] * r, wl[:128], preferred_element_type=jnp.float32)
    h += jnp.dot(ah_ref[...] * r, wl[128:], preferred_element_type=jnp.float32)
    h += jnp.dot(hlo_ref[...], wr[:128], preferred_element_type=jnp.float32)
    h += jnp.dot(hhi_ref[...], wr[128:], preferred_element_type=jnp.float32)
    h = jnp.maximum(h + bl_ref[...], 0.0)
    h = jnp.maximum(jnp.dot(h, w1_ref[...],
                            preferred_element_type=jnp.float32) + b1_ref[...],
                    0.0)
    h = jnp.maximum(jnp.dot(h, w2_ref[...],
                            preferred_element_type=jnp.float32) + b2_ref[...],
                    0.0)
    o_ref[...] = jnp.dot(h, w3_ref[...],
                         preferred_element_type=jnp.float32) + b3_ref[...]


_R = 1000


def _row_spec(w):
    return pl.BlockSpec((_R, w), lambda i: (i, 0))


def _full_spec(shape):
    return pl.BlockSpec(shape, lambda i: (0,) * len(shape))


def _tc0(aa, ab, ca, cb, x, wl, bl, wr):
    return pl.pallas_call(
        _tc0_body,
        grid=(N_NODES // _R,),
        in_specs=[
            _row_spec(128), _row_spec(128), _row_spec(128), _row_spec(128),
            _row_spec(128),
            _full_spec((128, 256)), _full_spec((1, 256)),
            _full_spec((128, 256)),
        ],
        out_specs=[_row_spec(128), _row_spec(128)],
        out_shape=[jax.ShapeDtypeStruct((N_NODES, 128), jnp.float32)] * 2,
    )(aa, ab, ca, cb, x, wl, bl, wr)


def _tc1(al, ah, ca, cb, hlo, hhi, wl, bl, wr, w1, b1, w2, b2, w3, b3):
    return pl.pallas_call(
        _tc1_body,
        grid=(N_NODES // _R,),
        in_specs=[
            _row_spec(128), _row_spec(128), _row_spec(128), _row_spec(128),
            _row_spec(128), _row_spec(128),
            _full_spec((256, 256)), _full_spec((1, 256)),
            _full_spec((256, 256)),
            _full_spec((256, 256)), _full_spec((1, 256)),
            _full_spec((256, 256)), _full_spec((1, 256)),
            _full_spec((256, 128)), _full_spec((1, 128)),
        ],
        out_specs=_row_spec(128),
        out_shape=jax.ShapeDtypeStruct((N_NODES, 128), jnp.float32),
    )(al, ah, ca, cb, hlo, hhi, wl, bl, wr, w1, b1, w2, b2, w3, b3)


def kernel(x, edge_index, Wl0, bl0, Wr0, Wl1, bl1, Wr1, W1, b1, W2, b2,
           W3, b3):
    i32 = jnp.int32
    src = edge_index[0].astype(i32)
    dst = edge_index[1].astype(i32)
    pad = E_PAD - N_EDGES
    src_p = jnp.concatenate([src, jnp.zeros((pad,), i32)]).reshape(
        NT * CH1, SUB)
    dst_p = jnp.concatenate([dst, jnp.full((pad,), N_NODES, i32)]).reshape(
        NT * CH1, SUB)

    zrow = jnp.zeros((RD, 128), jnp.float32)
    ones = jnp.ones((SUB, 128), jnp.float32)

    acc_a, acc_b = _agg_edge_split()(x, src_p, dst_p, zrow)
    cnt_a, cnt_b = _count_edges()(dst_p, zrow, ones)
    ca = cnt_a[:N_NODES]
    cb = cnt_b[:N_NODES]

    h_lo, h_hi = _tc0(acc_a[:N_NODES], acc_b[:N_NODES], ca, cb, x,
                      Wl0, bl0.reshape(1, -1), Wr0)

    agg1_lo, agg1_hi = _agg_feat_split()(h_lo, h_hi, src_p, dst_p, zrow)

    w3p = jnp.pad(W3, ((0, 0), (0, 128 - W3.shape[1])))
    b3p = jnp.pad(b3, (0, 128 - b3.shape[0])).reshape(1, -1)
    out = _tc1(agg1_lo[:N_NODES], agg1_hi[:N_NODES], ca, cb, h_lo, h_hi,
               Wl1, bl1.reshape(1, -1), Wr1, W1, b1.reshape(1, -1),
               W2, b2.reshape(1, -1), w3p, b3p)
    return out[:, :40]
